# Initial kernel scaffold; baseline (speedup 1.0000x reference)
import jax
import jax.numpy as jnp
from jax.experimental import pallas as pl


def kernel(x, edge_index, W, att_src, att_dst, bias):
    raise NotImplementedError("write your pallas kernel here")


# speedup vs baseline: 9.5530x; 9.5530x over previous
import functools

import jax
import jax.numpy as jnp
from jax import lax
from jax.experimental import pallas as pl
from jax.experimental.pallas import tpu as pltpu
from jax.experimental.pallas import tpu_sc as plsc

F32 = jnp.float32
NS = 16
NC = 2
NW = NC * NS


def _tc1_body(x_ref, w_ref, asv_ref, adv_ref, h_ref, a2_ref):
    h = jnp.dot(x_ref[...], w_ref[...], preferred_element_type=F32)
    h_ref[...] = h
    asr = lax.dot_general(asv_ref[...], h, (((1,), (1,)), ((), ())),
                          preferred_element_type=F32)
    adr = lax.dot_general(adv_ref[...], h, (((1,), (1,)), ((), ())),
                          preferred_element_type=F32)
    a2_ref[...] = jnp.concatenate(
        [asr, adr, jnp.zeros((6, asr.shape[1]), F32)], axis=0)


def _tc1(x_pad, W, att_src, att_dst, NPAD, D, Br):
    grid = (NPAD // Br,)
    return pl.pallas_call(
        _tc1_body,
        grid=grid,
        in_specs=[
            pl.BlockSpec((Br, D), lambda i: (i, 0)),
            pl.BlockSpec((D, D), lambda i: (0, 0)),
            pl.BlockSpec((1, D), lambda i: (0, 0)),
            pl.BlockSpec((1, D), lambda i: (0, 0)),
        ],
        out_specs=[
            pl.BlockSpec((Br, D), lambda i: (i, 0)),
            pl.BlockSpec((8, Br), lambda i: (0, i)),
        ],
        out_shape=[
            jax.ShapeDtypeStruct((NPAD, D), F32),
            jax.ShapeDtypeStruct((8, NPAD), F32),
        ],
    )(x_pad, W, att_src, att_dst)


def _sc_edge(src3, dst3, asrc, adst, h, NPAD, D, C, K):
    NR = NPAD // NS
    WC = 18
    mesh = plsc.VectorSubcoreMesh(core_axis_name="c", subcore_axis_name="s")

    @functools.partial(
        pl.kernel,
        out_type=(
            jax.ShapeDtypeStruct((NC, NPAD, D), F32),
            jax.ShapeDtypeStruct((NW, NPAD), F32),
        ),
        mesh=mesh,
        compiler_params=pltpu.CompilerParams(
            use_tc_tiling_on_sc=False, needs_layout_passes=False),
        scratch_types=[
            pltpu.VMEM((WC, K), jnp.int32),
            pltpu.VMEM((WC, K), jnp.int32),
            pltpu.VMEM((NPAD,), F32),
            pltpu.VMEM((NPAD,), F32),
            pltpu.VMEM((NPAD,), F32),
            pltpu.VMEM((K,), F32),
            pltpu.VMEM((K, D), F32),
            pltpu.VMEM_SHARED((NPAD, D), F32),
            pltpu.SemaphoreType.DMA,
        ],
    )
    def sc_kernel(src_hbm, dst_hbm, asrc_hbm, adst_hbm, h_hbm,
                  out_hbm, den_hbm,
                  src_win, dst_win, asrc_loc, adst_loc, den_loc,
                  ebuf, hbuf, out_sh, gsem):
        c = lax.axis_index("c")
        s = lax.axis_index("s")
        w = c * NS + s
        z16 = jnp.zeros((16,), F32)

        pltpu.sync_copy(asrc_hbm, asrc_loc)
        pltpu.sync_copy(adst_hbm, adst_loc)

        def zb(i, _):
            for q in range(D // 16):
                hbuf[i, pl.ds(q * 16, 16)] = z16
            return 0
        lax.fori_loop(0, K, zb, 0)

        def zd(i, _):
            den_loc[pl.ds(i * 16, 16)] = z16
            return 0
        lax.fori_loop(0, NPAD // 16, zd, 0)

        for t in range(NR // K):
            pltpu.sync_copy(hbuf, out_sh.at[pl.ds(s * NR + t * K, K)])
        plsc.subcore_barrier()

        def window(wi, _):
            pltpu.sync_copy(src_hbm.at[w, pl.ds(wi * WC, WC)], src_win)
            pltpu.sync_copy(dst_hbm.at[w, pl.ds(wi * WC, WC)], dst_win)

            def chunk(ci, _):
                pltpu.async_copy(h_hbm.at[src_win.at[ci]], hbuf, gsem).wait()

                def ew(j, _):
                    sv = src_win[ci, pl.ds(j * 16, 16)]
                    dv = dst_win[ci, pl.ds(j * 16, 16)]
                    av = (plsc.load_gather(asrc_loc, [sv])
                          + plsc.load_gather(adst_loc, [dv]))
                    av = jnp.where(av >= 0, av, av * F32(0.2))
                    ev = jnp.exp(av)
                    ebuf[pl.ds(j * 16, 16)] = ev
                    plsc.addupdate_scatter(den_loc, [dv], ev)
                    return 0
                lax.fori_loop(0, K // 16, ew, 0)

                def scale(jj, _):
                    ev = plsc.load_gather(
                        ebuf, [jnp.full((16,), jj, jnp.int32)])
                    for q in range(D // 16):
                        hbuf[jj, pl.ds(q * 16, 16)] = (
                            hbuf[jj, pl.ds(q * 16, 16)] * ev)
                    return 0
                lax.fori_loop(0, K, scale, 0)

                pltpu.sync_copy(hbuf, out_sh.at[dst_win.at[ci]], add=True)
                return 0
            lax.fori_loop(0, WC, chunk, 0)
            return 0
        lax.fori_loop(0, C // WC, window, 0)

        pltpu.sync_copy(den_loc, den_hbm.at[w])
        plsc.subcore_barrier()

        pltpu.sync_copy(out_sh.at[pl.ds(s * NR, NR)],
                        out_hbm.at[c, pl.ds(s * NR, NR)])

    return sc_kernel(src3, dst3, asrc, adst, h)


def _tc2_body(N, Br, out_ref, den_ref, x_ref, b_ref, u_ref, ssq_ref):
    i = pl.program_id(0)
    acc = out_ref[0] + out_ref[1]
    den = jnp.sum(den_ref[...], axis=0)
    o = acc / den + b_ref[...]
    u = jnp.where(o >= 0, o, o * F32(0.02)) + x_ref[...]
    rows = i * Br + lax.broadcasted_iota(jnp.int32, (Br, 1), 0)
    u = jnp.where(rows < N, u, F32(0.0))
    u_ref[...] = u

    @pl.when(i == 0)
    def _():
        ssq_ref[...] = jnp.zeros((1, 1), F32)
    ssq_ref[...] += jnp.sum(u * u).reshape(1, 1)


def _tc2(out_p, den_p, x_pad, bias, N, NPAD, D, Br):
    grid = (NPAD // Br,)
    return pl.pallas_call(
        functools.partial(_tc2_body, N, Br),
        grid=grid,
        in_specs=[
            pl.BlockSpec((2, Br, D), lambda i: (0, i, 0)),
            pl.BlockSpec((NW, Br, 1), lambda i: (0, i, 0)),
            pl.BlockSpec((Br, D), lambda i: (i, 0)),
            pl.BlockSpec((1, D), lambda i: (0, 0)),
        ],
        out_specs=[
            pl.BlockSpec((Br, D), lambda i: (i, 0)),
            pl.BlockSpec((1, 1), lambda i: (0, 0)),
        ],
        out_shape=[
            jax.ShapeDtypeStruct((NPAD, D), F32),
            jax.ShapeDtypeStruct((1, 1), F32),
        ],
    )(out_p, den_p, x_pad, bias)


def _tc3_body(a_ref, b_ref, s_ref, o_ref):
    p = lax.dot_general(a_ref[...], b_ref[...], (((1,), (1,)), ((), ())),
                        preferred_element_type=F32)
    o_ref[...] = p / s_ref[...]


def _tc3(u, ssq, N, D, Bi, Bj):
    gi = (N + Bi - 1) // Bi
    gj = (N + Bj - 1) // Bj
    return pl.pallas_call(
        _tc3_body,
        grid=(gi, gj),
        in_specs=[
            pl.BlockSpec((Bi, D), lambda i, j: (i, 0)),
            pl.BlockSpec((Bj, D), lambda i, j: (j, 0)),
            pl.BlockSpec((1, 1), lambda i, j: (0, 0)),
        ],
        out_specs=pl.BlockSpec((Bi, Bj), lambda i, j: (i, j)),
        out_shape=jax.ShapeDtypeStruct((N, N), F32),
    )(u, u, ssq)


def kernel(x, edge_index, W, att_src, att_dst, bias):
    N, D = x.shape
    E = edge_index.shape[1]
    NPAD = ((N + 1023) // 1024) * 1024
    K = 64
    Et = E + N
    EW = ((Et + NW - 1) // NW + K - 1) // K * K
    C = EW // K
    EPAD = NW * EW
    pad = EPAD - Et

    ei = edge_index.astype(jnp.int32)
    loop_idx = jnp.arange(N, dtype=jnp.int32)
    src = jnp.concatenate([ei[0], loop_idx, jnp.zeros((pad,), jnp.int32)])
    pad_dst = N + (jnp.arange(pad, dtype=jnp.int32) % (NPAD - N))
    dst = jnp.concatenate([ei[1], loop_idx, pad_dst])
    src3 = src.reshape(NW, C, K)
    dst3 = dst.reshape(NW, C, K)
    x_pad = jnp.pad(x, ((0, NPAD - N), (0, 0)))

    h_pad, a2 = _tc1(x_pad, W, att_src.reshape(1, D), att_dst.reshape(1, D),
                     NPAD, D, 1024)
    out_p, den_p = _sc_edge(src3, dst3, a2[0], a2[1], h_pad, NPAD, D, C, K)
    u, ssq = _tc2(out_p, den_p.reshape(NW, NPAD, 1), x_pad,
                  bias.reshape(1, D), N, NPAD, D, 1024)
    return _tc3(u, ssq, N, D, 512, 512)


# speedup vs baseline: 11.1062x; 1.1626x over previous
import functools

import jax
import jax.numpy as jnp
from jax import lax
from jax.experimental import pallas as pl
from jax.experimental.pallas import tpu as pltpu
from jax.experimental.pallas import tpu_sc as plsc

F32 = jnp.float32
NS = 16
NC = 2
NW = NC * NS


def _tc1_body(x_ref, w_ref, asv_ref, adv_ref, h_ref, a2_ref):
    h = jnp.dot(x_ref[...], w_ref[...], preferred_element_type=F32)
    h_ref[...] = h
    asr = lax.dot_general(asv_ref[...], h, (((1,), (1,)), ((), ())),
                          preferred_element_type=F32)
    adr = lax.dot_general(adv_ref[...], h, (((1,), (1,)), ((), ())),
                          preferred_element_type=F32)
    a2_ref[...] = jnp.concatenate(
        [asr, adr, jnp.zeros((6, asr.shape[1]), F32)], axis=0)


def _tc1(x_pad, W, att_src, att_dst, NPAD, D, Br):
    grid = (NPAD // Br,)
    return pl.pallas_call(
        _tc1_body,
        grid=grid,
        in_specs=[
            pl.BlockSpec((Br, D), lambda i: (i, 0)),
            pl.BlockSpec((D, D), lambda i: (0, 0)),
            pl.BlockSpec((1, D), lambda i: (0, 0)),
            pl.BlockSpec((1, D), lambda i: (0, 0)),
        ],
        out_specs=[
            pl.BlockSpec((Br, D), lambda i: (i, 0)),
            pl.BlockSpec((8, Br), lambda i: (0, i)),
        ],
        out_shape=[
            jax.ShapeDtypeStruct((NPAD, D), F32),
            jax.ShapeDtypeStruct((8, NPAD), F32),
        ],
    )(x_pad, W, att_src, att_dst)


def _sc_edge(src3, dst3, asrc, adst, h, NOUT, NPAD, D, C, K):
    NR = NOUT // NS
    WC = 12
    NWIN = C // WC
    assert NWIN % 2 == 0 and WC % 2 == 0
    mesh = plsc.VectorSubcoreMesh(core_axis_name="c", subcore_axis_name="s")

    @functools.partial(
        pl.kernel,
        out_type=(
            jax.ShapeDtypeStruct((NC, NOUT, D), F32),
            jax.ShapeDtypeStruct((NW, NOUT), F32),
        ),
        mesh=mesh,
        compiler_params=pltpu.CompilerParams(
            use_tc_tiling_on_sc=False, needs_layout_passes=False),
        scratch_types=[
            pltpu.VMEM((WC, K), jnp.int32),
            pltpu.VMEM((WC, K), jnp.int32),
            pltpu.VMEM((WC, K), jnp.int32),
            pltpu.VMEM((WC, K), jnp.int32),
            pltpu.VMEM((NPAD,), F32),
            pltpu.VMEM((NPAD,), F32),
            pltpu.VMEM((NOUT,), F32),
            pltpu.VMEM((K,), F32),
            pltpu.VMEM((K, D), F32),
            pltpu.VMEM((K, D), F32),
            pltpu.VMEM_SHARED((NOUT, D), F32),
            pltpu.SemaphoreType.DMA,
            pltpu.SemaphoreType.DMA,
            pltpu.SemaphoreType.DMA,
            pltpu.SemaphoreType.DMA,
            pltpu.SemaphoreType.DMA,
            pltpu.SemaphoreType.DMA,
        ],
    )
    def sc_kernel(src_hbm, dst_hbm, asrc_hbm, adst_hbm, h_hbm,
                  out_hbm, den_hbm,
                  src_wa, src_wb, dst_wa, dst_wb, asrc_loc, adst_loc,
                  den_loc, ebuf, hbufa, hbufb, out_sh,
                  wsema, wsemb, gsema, gsemb, ssema, ssemb):
        c = lax.axis_index("c")
        s = lax.axis_index("s")
        w = c * NS + s
        z16 = jnp.zeros((16,), F32)
        wres = [(src_wa, dst_wa, wsema), (src_wb, dst_wb, wsemb)]
        cres = [(hbufa, gsema, ssema), (hbufb, gsemb, ssemb)]

        pltpu.sync_copy(asrc_hbm, asrc_loc)
        pltpu.sync_copy(adst_hbm, adst_loc)

        def zb(i, _):
            for q in range(D // 16):
                hbufa[i, pl.ds(q * 16, 16)] = z16
            return 0
        lax.fori_loop(0, K, zb, 0)

        def zd(i, _):
            den_loc[pl.ds(i * 16, 16)] = z16
            return 0
        lax.fori_loop(0, NOUT // 16, zd, 0)

        for t in range(NR // K):
            pltpu.sync_copy(hbufa, out_sh.at[pl.ds(s * NR + t * K, K)])
        if NR % K:
            pltpu.sync_copy(hbufa.at[pl.ds(0, NR % K)],
                            out_sh.at[pl.ds(s * NR + (NR // K) * K, NR % K)])
        plsc.subcore_barrier()

        def win_load(g, wb):
            sw, dw, wsem = wres[wb]
            pltpu.async_copy(src_hbm.at[w, pl.ds(g * WC, WC)], sw, wsem)
            pltpu.async_copy(dst_hbm.at[w, pl.ds(g * WC, WC)], dw, wsem)

        def win_wait(g, wb):
            sw, dw, wsem = wres[wb]
            pltpu.make_async_copy(
                src_hbm.at[w, pl.ds(g * WC, WC)], sw, wsem).wait()
            pltpu.make_async_copy(
                dst_hbm.at[w, pl.ds(g * WC, WC)], dw, wsem).wait()

        def gather_start(sw, ci, cb):
            buf, gsem, _ = cres[cb]
            pltpu.async_copy(h_hbm.at[sw.at[ci]], buf, gsem)

        def gather_wait(sw, ci, cb):
            buf, gsem, _ = cres[cb]
            pltpu.make_async_copy(h_hbm.at[sw.at[ci]], buf, gsem).wait()

        def do_chunk(sw, dw, ci, cb):
            buf, _, ssem = cres[cb]
            for j in range(K // 16):
                sv = sw[ci, pl.ds(j * 16, 16)]
                dv = dw[ci, pl.ds(j * 16, 16)]
                av = (plsc.load_gather(asrc_loc, [sv])
                      + plsc.load_gather(adst_loc, [dv]))
                av = jnp.where(av >= 0, av, av * F32(0.2))
                ev = jnp.exp(av)
                ebuf[pl.ds(j * 16, 16)] = ev
                plsc.addupdate_scatter(den_loc, [dv], ev)

            gather_wait(sw, ci, cb)

            def scale(jj, _):
                ev = plsc.load_gather(ebuf, [jnp.full((16,), jj, jnp.int32)])
                for q in range(D // 16):
                    buf[jj, pl.ds(q * 16, 16)] = buf[jj, pl.ds(q * 16, 16)] * ev
                return 0
            lax.fori_loop(0, K, scale, 0, unroll=4)

            pltpu.async_copy(buf, out_sh.at[dw.at[ci]], ssem, add=True)
            pltpu.make_async_copy(buf, out_sh.at[dw.at[ci]], ssem).wait()

        def window_body(g, wb, prefetch):
            sw, dw, _ = wres[wb]
            win_wait(g, wb)
            gather_start(sw, 0, 0)
            gather_start(sw, 1, 1)

            def cpair(cp, _):
                for cb in range(2):
                    ci = 2 * cp + cb
                    do_chunk(sw, dw, ci, cb)
                    gather_start(sw, ci + 2, cb)
                return 0
            lax.fori_loop(0, WC // 2 - 1, cpair, 0)
            for cb in range(2):
                do_chunk(sw, dw, WC - 2 + cb, cb)
            if prefetch:
                win_load(g + 2, wb)

        win_load(0, 0)
        win_load(1, 1)

        def wpair(i, _):
            window_body(2 * i, 0, True)
            window_body(2 * i + 1, 1, True)
            return 0
        lax.fori_loop(0, NWIN // 2 - 1, wpair, 0)
        window_body(NWIN - 2, 0, False)
        window_body(NWIN - 1, 1, False)

        pltpu.sync_copy(den_loc, den_hbm.at[w])
        plsc.subcore_barrier()

        pltpu.sync_copy(out_sh.at[pl.ds(s * NR, NR)],
                        out_hbm.at[c, pl.ds(s * NR, NR)])

    return sc_kernel(src3, dst3, asrc, adst, h)


def _tc2_body(N, Br, out_ref, den_ref, x_ref, b_ref, u_ref, ssq_ref):
    i = pl.program_id(0)
    acc = out_ref[0] + out_ref[1]
    den = jnp.sum(den_ref[...], axis=0)
    o = acc / den + b_ref[...]
    u = jnp.where(o >= 0, o, o * F32(0.02)) + x_ref[...]
    rows = i * Br + lax.broadcasted_iota(jnp.int32, (Br, 1), 0)
    u = jnp.where(rows < N, u, F32(0.0))
    u_ref[...] = u

    @pl.when(i == 0)
    def _():
        ssq_ref[...] = jnp.zeros((1, 1), F32)
    ssq_ref[...] += jnp.sum(u * u).reshape(1, 1)


def _tc2(out_p, den_p, x_pad, bias, N, NPAD, D, Br):
    grid = (NPAD // Br,)
    return pl.pallas_call(
        functools.partial(_tc2_body, N, Br),
        grid=grid,
        in_specs=[
            pl.BlockSpec((2, Br, D), lambda i: (0, i, 0)),
            pl.BlockSpec((NW, Br, 1), lambda i: (0, i, 0)),
            pl.BlockSpec((Br, D), lambda i: (i, 0)),
            pl.BlockSpec((1, D), lambda i: (0, 0)),
        ],
        out_specs=[
            pl.BlockSpec((Br, D), lambda i: (i, 0)),
            pl.BlockSpec((1, 1), lambda i: (0, 0)),
        ],
        out_shape=[
            jax.ShapeDtypeStruct((NPAD, D), F32),
            jax.ShapeDtypeStruct((1, 1), F32),
        ],
    )(out_p, den_p, x_pad, bias)


def _tc3_body(a_ref, b_ref, s_ref, o_ref):
    p = lax.dot_general(a_ref[...], b_ref[...], (((1,), (1,)), ((), ())),
                        preferred_element_type=F32)
    o_ref[...] = p / s_ref[...]


def _tc3(u, ssq, N, D, Bi, Bj):
    gi = (N + Bi - 1) // Bi
    gj = (N + Bj - 1) // Bj
    return pl.pallas_call(
        _tc3_body,
        grid=(gi, gj),
        in_specs=[
            pl.BlockSpec((Bi, D), lambda i, j: (i, 0)),
            pl.BlockSpec((Bj, D), lambda i, j: (j, 0)),
            pl.BlockSpec((1, 1), lambda i, j: (0, 0)),
        ],
        out_specs=pl.BlockSpec((Bi, Bj), lambda i, j: (i, j)),
        out_shape=jax.ShapeDtypeStruct((N, N), F32),
    )(u, u, ssq)


def kernel(x, edge_index, W, att_src, att_dst, bias):
    N, D = x.shape
    E = edge_index.shape[1]
    NPAD = ((N + 1023) // 1024) * 1024
    NOUT = ((N + 15) // 16) * 16 + 48
    K = 48
    WK = 12 * K
    Et = E + N
    EW = ((Et + NW - 1) // NW + 2 * WK - 1) // (2 * WK) * (2 * WK)
    C = EW // K
    EPAD = NW * EW
    pad = EPAD - Et

    ei = edge_index.astype(jnp.int32)
    loop_idx = jnp.arange(N, dtype=jnp.int32)
    src = jnp.concatenate([ei[0], loop_idx, jnp.zeros((pad,), jnp.int32)])
    pad_dst = N + (jnp.arange(pad, dtype=jnp.int32) % (NOUT - N))
    dst = jnp.concatenate([ei[1], loop_idx, pad_dst])
    src3 = src.reshape(NW, C, K)
    dst3 = dst.reshape(NW, C, K)
    x_pad = jnp.pad(x, ((0, NPAD - N), (0, 0)))

    h_pad, a2 = _tc1(x_pad, W, att_src.reshape(1, D), att_dst.reshape(1, D),
                     NPAD, D, 1024)
    out_p, den_p = _sc_edge(src3, dst3, a2[0], a2[1], h_pad,
                            NOUT, NPAD, D, C, K)
    u, ssq = _tc2(out_p, den_p.reshape(NW, NOUT, 1), x_pad,
                  bias.reshape(1, D), N, NPAD, D, 1024)
    return _tc3(u, ssq, N, D, 512, 512)


# speedup vs baseline: 14.9288x; 1.3442x over previous
import functools

import jax
import jax.numpy as jnp
from jax import lax
from jax.experimental import pallas as pl
from jax.experimental.pallas import tpu as pltpu
from jax.experimental.pallas import tpu_sc as plsc

F32 = jnp.float32
NS = 16
NC = 2
NW = NC * NS


def _tc1_body(x_ref, w_ref, asv_ref, adv_ref, h_ref, a2_ref):
    h = jnp.dot(x_ref[...], w_ref[...], preferred_element_type=F32)
    h_ref[...] = h
    asr = lax.dot_general(asv_ref[...], h, (((1,), (1,)), ((), ())),
                          preferred_element_type=F32)
    adr = lax.dot_general(adv_ref[...], h, (((1,), (1,)), ((), ())),
                          preferred_element_type=F32)
    a2_ref[...] = jnp.concatenate(
        [asr, adr, jnp.zeros((6, asr.shape[1]), F32)], axis=0)


def _tc1(x_pad, W, att_src, att_dst, NPAD, D, Br):
    grid = (NPAD // Br,)
    return pl.pallas_call(
        _tc1_body,
        grid=grid,
        in_specs=[
            pl.BlockSpec((Br, D), lambda i: (i, 0)),
            pl.BlockSpec((D, D), lambda i: (0, 0)),
            pl.BlockSpec((1, D), lambda i: (0, 0)),
            pl.BlockSpec((1, D), lambda i: (0, 0)),
        ],
        out_specs=[
            pl.BlockSpec((Br, D), lambda i: (i, 0)),
            pl.BlockSpec((8, Br), lambda i: (0, i)),
        ],
        out_shape=[
            jax.ShapeDtypeStruct((NPAD, D), F32),
            jax.ShapeDtypeStruct((8, NPAD), F32),
        ],
    )(x_pad, W, att_src, att_dst)


def _sc_edge(src3, dst3, asrc, adst, h, NOUT, NPAD, D, C, K):
    NR = NOUT // NS
    WC = 12
    NWIN = C // WC
    assert NWIN % 2 == 0 and WC % 2 == 0
    mesh = plsc.VectorSubcoreMesh(core_axis_name="c", subcore_axis_name="s")

    @functools.partial(
        pl.kernel,
        out_type=(
            jax.ShapeDtypeStruct((NC, NOUT, D), F32),
            jax.ShapeDtypeStruct((NW, NOUT), F32),
        ),
        mesh=mesh,
        compiler_params=pltpu.CompilerParams(
            use_tc_tiling_on_sc=False, needs_layout_passes=False),
        scratch_types=[
            pltpu.VMEM((WC, K), jnp.int32),
            pltpu.VMEM((WC, K), jnp.int32),
            pltpu.VMEM((WC, K), jnp.int32),
            pltpu.VMEM((WC, K), jnp.int32),
            pltpu.VMEM((NPAD,), F32),
            pltpu.VMEM((NPAD,), F32),
            pltpu.VMEM((NOUT,), F32),
            pltpu.VMEM((K,), F32),
            pltpu.VMEM((K, D), F32),
            pltpu.VMEM((K, D), F32),
            pltpu.VMEM_SHARED((NOUT, D), F32),
            pltpu.SemaphoreType.DMA,
            pltpu.SemaphoreType.DMA,
            pltpu.SemaphoreType.DMA,
            pltpu.SemaphoreType.DMA,
            pltpu.SemaphoreType.DMA,
            pltpu.SemaphoreType.DMA,
        ],
    )
    def sc_kernel(src_hbm, dst_hbm, asrc_hbm, adst_hbm, h_hbm,
                  out_hbm, den_hbm,
                  src_wa, src_wb, dst_wa, dst_wb, asrc_loc, adst_loc,
                  den_loc, ebuf, hbufa, hbufb, out_sh,
                  wsema, wsemb, gsema, gsemb, ssema, ssemb):
        c = lax.axis_index("c")
        s = lax.axis_index("s")
        w = c * NS + s
        z16 = jnp.zeros((16,), F32)
        wres = [(src_wa, dst_wa, wsema), (src_wb, dst_wb, wsemb)]
        cres = [(hbufa, gsema, ssema), (hbufb, gsemb, ssemb)]

        pltpu.sync_copy(asrc_hbm, asrc_loc)
        pltpu.sync_copy(adst_hbm, adst_loc)

        def zb(i, _):
            for q in range(D // 16):
                hbufa[i, pl.ds(q * 16, 16)] = z16
            return 0
        lax.fori_loop(0, K, zb, 0)

        def zd(i, _):
            den_loc[pl.ds(i * 16, 16)] = z16
            return 0
        lax.fori_loop(0, NOUT // 16, zd, 0)

        for t in range(NR // K):
            pltpu.sync_copy(hbufa, out_sh.at[pl.ds(s * NR + t * K, K)])
        if NR % K:
            pltpu.sync_copy(hbufa.at[pl.ds(0, NR % K)],
                            out_sh.at[pl.ds(s * NR + (NR // K) * K, NR % K)])
        plsc.subcore_barrier()

        def win_load(g, wb):
            sw, dw, wsem = wres[wb]
            pltpu.async_copy(src_hbm.at[w, pl.ds(g * WC, WC)], sw, wsem)
            pltpu.async_copy(dst_hbm.at[w, pl.ds(g * WC, WC)], dw, wsem)

        def win_wait(g, wb):
            sw, dw, wsem = wres[wb]
            pltpu.make_async_copy(
                src_hbm.at[w, pl.ds(g * WC, WC)], sw, wsem).wait()
            pltpu.make_async_copy(
                dst_hbm.at[w, pl.ds(g * WC, WC)], dw, wsem).wait()

        def gather_start(sw, ci, cb):
            buf, gsem, _ = cres[cb]
            pltpu.async_copy(h_hbm.at[sw.at[ci]], buf, gsem)

        def gather_wait(sw, ci, cb):
            buf, gsem, _ = cres[cb]
            pltpu.make_async_copy(h_hbm.at[sw.at[ci]], buf, gsem).wait()

        def do_chunk(sw, dw, ci, cb):
            buf, _, ssem = cres[cb]
            for j in range(K // 16):
                sv = sw[ci, pl.ds(j * 16, 16)]
                dv = dw[ci, pl.ds(j * 16, 16)]
                av = (plsc.load_gather(asrc_loc, [sv])
                      + plsc.load_gather(adst_loc, [dv]))
                av = jnp.where(av >= 0, av, av * F32(0.2))
                ev = jnp.exp(av)
                ebuf[pl.ds(j * 16, 16)] = ev
                plsc.addupdate_scatter(den_loc, [dv], ev)

            gather_wait(sw, ci, cb)

            def scale(jj, _):
                ev = plsc.load_gather(ebuf, [jnp.full((16,), jj, jnp.int32)])
                for q in range(D // 16):
                    buf[jj, pl.ds(q * 16, 16)] = buf[jj, pl.ds(q * 16, 16)] * ev
                return 0
            lax.fori_loop(0, K, scale, 0, unroll=4)

            pltpu.async_copy(buf, out_sh.at[dw.at[ci]], ssem, add=True)
            pltpu.make_async_copy(buf, out_sh.at[dw.at[ci]], ssem).wait()

        def window_body(g, wb, prefetch):
            sw, dw, _ = wres[wb]
            win_wait(g, wb)
            gather_start(sw, 0, 0)
            gather_start(sw, 1, 1)

            def cpair(cp, _):
                for cb in range(2):
                    ci = 2 * cp + cb
                    do_chunk(sw, dw, ci, cb)
                    gather_start(sw, ci + 2, cb)
                return 0
            lax.fori_loop(0, WC // 2 - 1, cpair, 0)
            for cb in range(2):
                do_chunk(sw, dw, WC - 2 + cb, cb)
            if prefetch:
                win_load(g + 2, wb)

        win_load(0, 0)
        win_load(1, 1)

        def wpair(i, _):
            window_body(2 * i, 0, True)
            window_body(2 * i + 1, 1, True)
            return 0
        lax.fori_loop(0, NWIN // 2 - 1, wpair, 0)
        window_body(NWIN - 2, 0, False)
        window_body(NWIN - 1, 1, False)

        pltpu.sync_copy(den_loc, den_hbm.at[w])
        plsc.subcore_barrier()

        pltpu.sync_copy(out_sh.at[pl.ds(s * NR, NR)],
                        out_hbm.at[c, pl.ds(s * NR, NR)])

    return sc_kernel(src3, dst3, asrc, adst, h)


def _tc2_body(N, Br, out_ref, den_ref, x_ref, b_ref, u_ref, ssq_ref):
    i = pl.program_id(0)
    acc = out_ref[0] + out_ref[1]
    den = jnp.sum(den_ref[...], axis=0)
    o = acc / den + b_ref[...]
    u = jnp.where(o >= 0, o, o * F32(0.02)) + x_ref[...]
    rows = i * Br + lax.broadcasted_iota(jnp.int32, (Br, 1), 0)
    u = jnp.where(rows < N, u, F32(0.0))
    u_ref[...] = u

    @pl.when(i == 0)
    def _():
        ssq_ref[...] = jnp.zeros((1, 1), F32)
    ssq_ref[...] += jnp.sum(u * u).reshape(1, 1)


def _tc2(out_p, den_p, x_pad, bias, N, NPAD, D, Br):
    grid = (NPAD // Br,)
    return pl.pallas_call(
        functools.partial(_tc2_body, N, Br),
        grid=grid,
        in_specs=[
            pl.BlockSpec((2, Br, D), lambda i: (0, i, 0)),
            pl.BlockSpec((NW, Br, 1), lambda i: (0, i, 0)),
            pl.BlockSpec((Br, D), lambda i: (i, 0)),
            pl.BlockSpec((1, D), lambda i: (0, 0)),
        ],
        out_specs=[
            pl.BlockSpec((Br, D), lambda i: (i, 0)),
            pl.BlockSpec((1, 1), lambda i: (0, 0)),
        ],
        out_shape=[
            jax.ShapeDtypeStruct((NPAD, D), F32),
            jax.ShapeDtypeStruct((1, 1), F32),
        ],
    )(out_p, den_p, x_pad, bias)


def _tc3_body(a_ref, b_ref, s_ref, o_ref):
    p = lax.dot_general(a_ref[...], b_ref[...], (((1,), (1,)), ((), ())),
                        preferred_element_type=F32)
    o_ref[...] = p / s_ref[...]


def _tc3(u, ssq, N, D, Bi, Bj):
    gi = (N + Bi - 1) // Bi
    gj = (N + Bj - 1) // Bj
    return pl.pallas_call(
        _tc3_body,
        grid=(gj, gi),
        in_specs=[
            pl.BlockSpec((Bi, D), lambda j, i: (i, 0)),
            pl.BlockSpec((Bj, D), lambda j, i: (j, 0)),
            pl.BlockSpec((1, 1), lambda j, i: (0, 0)),
        ],
        out_specs=pl.BlockSpec((Bi, Bj), lambda j, i: (i, j)),
        out_shape=jax.ShapeDtypeStruct((N, N), F32),
    )(u, u, ssq)


def kernel(x, edge_index, W, att_src, att_dst, bias):
    N, D = x.shape
    E = edge_index.shape[1]
    NPAD = ((N + 1023) // 1024) * 1024
    NOUT = ((N + 15) // 16) * 16 + 48
    K = 48
    WK = 12 * K
    Et = E + N
    EW = ((Et + NW - 1) // NW + 2 * WK - 1) // (2 * WK) * (2 * WK)
    C = EW // K
    EPAD = NW * EW
    pad = EPAD - Et

    ei = edge_index.astype(jnp.int32)
    loop_idx = jnp.arange(N, dtype=jnp.int32)
    src = jnp.concatenate([ei[0], loop_idx, jnp.zeros((pad,), jnp.int32)])
    pad_dst = N + (jnp.arange(pad, dtype=jnp.int32) % (NOUT - N))
    dst = jnp.concatenate([ei[1], loop_idx, pad_dst])
    src3 = src.reshape(NW, C, K)
    dst3 = dst.reshape(NW, C, K)
    x_pad = jnp.pad(x, ((0, NPAD - N), (0, 0)))

    h_pad, a2 = _tc1(x_pad, W, att_src.reshape(1, D), att_dst.reshape(1, D),
                     NPAD, D, 1024)
    out_p, den_p = _sc_edge(src3, dst3, a2[0], a2[1], h_pad,
                            NOUT, NPAD, D, C, K)
    u, ssq = _tc2(out_p, den_p.reshape(NW, NOUT, 1), x_pad,
                  bias.reshape(1, D), N, NPAD, D, 1024)
    return _tc3(u, ssq, N, D, 512, 5120)


# speedup vs baseline: 15.2646x; 1.0225x over previous
import functools

import jax
import jax.numpy as jnp
from jax import lax
from jax.experimental import pallas as pl
from jax.experimental.pallas import tpu as pltpu
from jax.experimental.pallas import tpu_sc as plsc

F32 = jnp.float32
NS = 16
NC = 2
NW = NC * NS


def _tc1_body(x_ref, w_ref, asv_ref, adv_ref, h_ref, a2_ref):
    h = jnp.dot(x_ref[...], w_ref[...], preferred_element_type=F32)
    h_ref[...] = h
    asr = lax.dot_general(asv_ref[...], h, (((1,), (1,)), ((), ())),
                          preferred_element_type=F32)
    adr = lax.dot_general(adv_ref[...], h, (((1,), (1,)), ((), ())),
                          preferred_element_type=F32)
    a2_ref[...] = jnp.concatenate(
        [asr, adr, jnp.zeros((6, asr.shape[1]), F32)], axis=0)


def _tc1(x_pad, W, att_src, att_dst, NPAD, D, Br):
    grid = (NPAD // Br,)
    return pl.pallas_call(
        _tc1_body,
        grid=grid,
        in_specs=[
            pl.BlockSpec((Br, D), lambda i: (i, 0)),
            pl.BlockSpec((D, D), lambda i: (0, 0)),
            pl.BlockSpec((1, D), lambda i: (0, 0)),
            pl.BlockSpec((1, D), lambda i: (0, 0)),
        ],
        out_specs=[
            pl.BlockSpec((Br, D), lambda i: (i, 0)),
            pl.BlockSpec((8, Br), lambda i: (0, i)),
        ],
        out_shape=[
            jax.ShapeDtypeStruct((NPAD, D), F32),
            jax.ShapeDtypeStruct((8, NPAD), F32),
        ],
    )(x_pad, W, att_src, att_dst)


def _sc_edge(src3, dst3, asrc, adst, h, NOUT, NPAD, D, C, K):
    NR = NOUT // NS
    WC = 12
    NWIN = C // WC
    assert NWIN % 2 == 0 and WC % 2 == 0
    mesh = plsc.VectorSubcoreMesh(core_axis_name="c", subcore_axis_name="s")

    @functools.partial(
        pl.kernel,
        out_type=(
            jax.ShapeDtypeStruct((NC, NOUT, D), F32),
            jax.ShapeDtypeStruct((NW, NOUT), F32),
        ),
        mesh=mesh,
        compiler_params=pltpu.CompilerParams(
            use_tc_tiling_on_sc=False, needs_layout_passes=False),
        scratch_types=[
            pltpu.VMEM((WC, K), jnp.int32),
            pltpu.VMEM((WC, K), jnp.int32),
            pltpu.VMEM((WC, K), jnp.int32),
            pltpu.VMEM((WC, K), jnp.int32),
            pltpu.VMEM((NPAD,), F32),
            pltpu.VMEM((NPAD,), F32),
            pltpu.VMEM((NOUT,), F32),
            pltpu.VMEM((K, D), F32),
            pltpu.VMEM((K, D), F32),
            pltpu.VMEM_SHARED((NOUT, D), F32),
            pltpu.SemaphoreType.DMA,
            pltpu.SemaphoreType.DMA,
            pltpu.SemaphoreType.DMA,
            pltpu.SemaphoreType.DMA,
            pltpu.SemaphoreType.DMA,
            pltpu.SemaphoreType.DMA,
        ],
    )
    def sc_kernel(src_hbm, dst_hbm, asrc_hbm, adst_hbm, h_hbm,
                  out_hbm, den_hbm,
                  src_wa, src_wb, dst_wa, dst_wb, asrc_loc, adst_loc,
                  den_loc, hbufa, hbufb, out_sh,
                  wsema, wsemb, gsema, gsemb, ssema, ssemb):
        c = lax.axis_index("c")
        s = lax.axis_index("s")
        w = c * NS + s
        z16 = jnp.zeros((16,), F32)
        wres = [(src_wa, dst_wa, wsema), (src_wb, dst_wb, wsemb)]
        cres = [(hbufa, gsema, ssema), (hbufb, gsemb, ssemb)]

        pltpu.sync_copy(asrc_hbm, asrc_loc)
        pltpu.sync_copy(adst_hbm, adst_loc)

        def zb(i, _):
            for q in range(D // 16):
                hbufa[i, pl.ds(q * 16, 16)] = z16
            return 0
        lax.fori_loop(0, K, zb, 0)

        def zd(i, _):
            den_loc[pl.ds(i * 16, 16)] = z16
            return 0
        lax.fori_loop(0, NOUT // 16, zd, 0)

        for t in range(NR // K):
            pltpu.sync_copy(hbufa, out_sh.at[pl.ds(s * NR + t * K, K)])
        if NR % K:
            pltpu.sync_copy(hbufa.at[pl.ds(0, NR % K)],
                            out_sh.at[pl.ds(s * NR + (NR // K) * K, NR % K)])
        plsc.subcore_barrier()

        def win_load(g, wb):
            sw, dw, wsem = wres[wb]
            pltpu.async_copy(src_hbm.at[w, pl.ds(g * WC, WC)], sw, wsem)
            pltpu.async_copy(dst_hbm.at[w, pl.ds(g * WC, WC)], dw, wsem)

        def win_wait(g, wb):
            sw, dw, wsem = wres[wb]
            pltpu.make_async_copy(
                src_hbm.at[w, pl.ds(g * WC, WC)], sw, wsem).wait()
            pltpu.make_async_copy(
                dst_hbm.at[w, pl.ds(g * WC, WC)], dw, wsem).wait()

        def gather_start(sw, ci, cb):
            buf, gsem, _ = cres[cb]
            pltpu.async_copy(h_hbm.at[sw.at[ci]], buf, gsem)

        def gather_wait(sw, ci, cb):
            buf, gsem, _ = cres[cb]
            pltpu.make_async_copy(h_hbm.at[sw.at[ci]], buf, gsem).wait()

        def chunk_exp(sw, dw, ci):
            evs = []
            for j in range(K // 16):
                sv = sw[ci, pl.ds(j * 16, 16)]
                dv = dw[ci, pl.ds(j * 16, 16)]
                av = (plsc.load_gather(asrc_loc, [sv])
                      + plsc.load_gather(adst_loc, [dv]))
                av = jnp.where(av >= 0, av, av * F32(0.2))
                ev = jnp.exp(av)
                plsc.addupdate_scatter(den_loc, [dv], ev)
                evs.append(ev)
            return evs

        def chunk_main(sw, dw, ci, cb, evs):
            buf, _, ssem = cres[cb]
            gather_wait(sw, ci, cb)
            for j in range(K // 16):
                for t in range(16):
                    e = evs[j][t]
                    r = j * 16 + t
                    for q in range(D // 16):
                        buf[r, pl.ds(q * 16, 16)] = buf[r, pl.ds(q * 16, 16)] * e
            pltpu.async_copy(buf, out_sh.at[dw.at[ci]], ssem, add=True)

        def scatter_wait(dw, ci, cb):
            buf, _, ssem = cres[cb]
            pltpu.make_async_copy(buf, out_sh.at[dw.at[ci]], ssem).wait()

        def window_body(g, wb, prefetch):
            sw, dw, _ = wres[wb]
            win_wait(g, wb)
            gather_start(sw, 0, 0)
            gather_start(sw, 1, 1)

            def cpair(cp, _):
                a = 2 * cp
                b = a + 1
                evs_a = chunk_exp(sw, dw, a)
                chunk_main(sw, dw, a, 0, evs_a)
                evs_b = chunk_exp(sw, dw, b)
                scatter_wait(dw, a, 0)
                gather_start(sw, a + 2, 0)
                chunk_main(sw, dw, b, 1, evs_b)
                scatter_wait(dw, b, 1)
                gather_start(sw, b + 2, 1)
                return 0
            lax.fori_loop(0, WC // 2 - 1, cpair, 0)
            a = WC - 2
            evs_a = chunk_exp(sw, dw, a)
            chunk_main(sw, dw, a, 0, evs_a)
            evs_b = chunk_exp(sw, dw, a + 1)
            scatter_wait(dw, a, 0)
            chunk_main(sw, dw, a + 1, 1, evs_b)
            scatter_wait(dw, a + 1, 1)
            if prefetch:
                win_load(g + 2, wb)

        win_load(0, 0)
        win_load(1, 1)

        def wpair(i, _):
            window_body(2 * i, 0, True)
            window_body(2 * i + 1, 1, True)
            return 0
        lax.fori_loop(0, NWIN // 2 - 1, wpair, 0)
        window_body(NWIN - 2, 0, False)
        window_body(NWIN - 1, 1, False)

        pltpu.sync_copy(den_loc, den_hbm.at[w])
        plsc.subcore_barrier()

        pltpu.sync_copy(out_sh.at[pl.ds(s * NR, NR)],
                        out_hbm.at[c, pl.ds(s * NR, NR)])

    return sc_kernel(src3, dst3, asrc, adst, h)


def _tc2_body(N, Br, out_ref, den_ref, x_ref, b_ref, u_ref, ssq_ref):
    i = pl.program_id(0)
    acc = out_ref[0] + out_ref[1]
    den = jnp.sum(den_ref[...], axis=0)
    o = acc / den + b_ref[...]
    u = jnp.where(o >= 0, o, o * F32(0.02)) + x_ref[...]
    rows = i * Br + lax.broadcasted_iota(jnp.int32, (Br, 1), 0)
    u = jnp.where(rows < N, u, F32(0.0))
    u_ref[...] = u

    @pl.when(i == 0)
    def _():
        ssq_ref[...] = jnp.zeros((1, 1), F32)
    ssq_ref[...] += jnp.sum(u * u).reshape(1, 1)


def _tc2(out_p, den_p, x_pad, bias, N, NPAD, D, Br):
    grid = (NPAD // Br,)
    return pl.pallas_call(
        functools.partial(_tc2_body, N, Br),
        grid=grid,
        in_specs=[
            pl.BlockSpec((2, Br, D), lambda i: (0, i, 0)),
            pl.BlockSpec((NW, Br, 1), lambda i: (0, i, 0)),
            pl.BlockSpec((Br, D), lambda i: (i, 0)),
            pl.BlockSpec((1, D), lambda i: (0, 0)),
        ],
        out_specs=[
            pl.BlockSpec((Br, D), lambda i: (i, 0)),
            pl.BlockSpec((1, 1), lambda i: (0, 0)),
        ],
        out_shape=[
            jax.ShapeDtypeStruct((NPAD, D), F32),
            jax.ShapeDtypeStruct((1, 1), F32),
        ],
    )(out_p, den_p, x_pad, bias)


def _tc3_body(a_ref, b_ref, s_ref, o_ref):
    p = lax.dot_general(a_ref[...], b_ref[...], (((1,), (1,)), ((), ())),
                        preferred_element_type=F32)
    o_ref[...] = p / s_ref[...]


def _tc3(u, ssq, N, D, Bi, Bj):
    gi = (N + Bi - 1) // Bi
    gj = (N + Bj - 1) // Bj
    return pl.pallas_call(
        _tc3_body,
        grid=(gj, gi),
        in_specs=[
            pl.BlockSpec((Bi, D), lambda j, i: (i, 0)),
            pl.BlockSpec((Bj, D), lambda j, i: (j, 0)),
            pl.BlockSpec((1, 1), lambda j, i: (0, 0)),
        ],
        out_specs=pl.BlockSpec((Bi, Bj), lambda j, i: (i, j)),
        out_shape=jax.ShapeDtypeStruct((N, N), F32),
    )(u, u, ssq)


def kernel(x, edge_index, W, att_src, att_dst, bias):
    N, D = x.shape
    E = edge_index.shape[1]
    NPAD = ((N + 1023) // 1024) * 1024
    NOUT = ((N + 15) // 16) * 16 + 48
    K = 48
    WK = 12 * K
    Et = E + N
    EW = ((Et + NW - 1) // NW + 2 * WK - 1) // (2 * WK) * (2 * WK)
    C = EW // K
    EPAD = NW * EW
    pad = EPAD - Et

    ei = edge_index.astype(jnp.int32)
    loop_idx = jnp.arange(N, dtype=jnp.int32)
    src = jnp.concatenate([ei[0], loop_idx, jnp.zeros((pad,), jnp.int32)])
    pad_dst = N + (jnp.arange(pad, dtype=jnp.int32) % (NOUT - N))
    dst = jnp.concatenate([ei[1], loop_idx, pad_dst])
    src3 = src.reshape(NW, C, K)
    dst3 = dst.reshape(NW, C, K)
    x_pad = jnp.pad(x, ((0, NPAD - N), (0, 0)))

    h_pad, a2 = _tc1(x_pad, W, att_src.reshape(1, D), att_dst.reshape(1, D),
                     NPAD, D, 1024)
    out_p, den_p = _sc_edge(src3, dst3, a2[0], a2[1], h_pad,
                            NOUT, NPAD, D, C, K)
    u, ssq = _tc2(out_p, den_p.reshape(NW, NOUT, 1), x_pad,
                  bias.reshape(1, D), N, NPAD, D, 1024)
    return _tc3(u, ssq, N, D, 512, 5120)
